# Initial kernel scaffold; baseline (speedup 1.0000x reference)
import jax
import jax.numpy as jnp
from jax.experimental import pallas as pl


def kernel(inputs_q, embed_table):
    raise NotImplementedError("write your pallas kernel here")


# speedup vs baseline: 218.4377x; 218.4377x over previous
import functools

import jax
import jax.numpy as jnp
from jax.experimental import pallas as pl
from jax.experimental.pallas import tpu as pltpu

_NUM_BUCKETS = 32
_MAX_DISTANCE = 128


def _diag_kernel(table_t_ref, diag_ref, *, length):
    width = diag_ref.shape[1]
    j = jax.lax.broadcasted_iota(jnp.int32, (1, width), 1)
    n = (length - 1) - j
    half = _NUM_BUCKETS // 2
    ret = jnp.where(n < 0, half, 0)
    n_abs = jnp.abs(n)
    max_exact = half // 2
    n_f = n_abs.astype(jnp.float32)
    val_if_large = max_exact + (
        jnp.log(n_f / max_exact + jnp.finfo(jnp.float32).eps)
        / jnp.log(_MAX_DISTANCE / max_exact)
        * (half - max_exact)
    ).astype(jnp.int32)
    val_if_large = jnp.minimum(val_if_large, half - 1)
    bucket = ret + jnp.where(n_abs < max_exact, n_abs, val_if_large)
    b_iota = jax.lax.broadcasted_iota(jnp.int32, (_NUM_BUCKETS, width), 0)
    onehot = (b_iota == bucket).astype(jnp.float32)
    diag_ref[...] = jnp.dot(
        table_t_ref[...], onehot, preferred_element_type=jnp.float32
    )


_WIN = 2304


_WIN = 2304


def _expand_kernel(diag_ref, out_ref, *, tq):
    heads = out_ref.shape[1]
    length = out_ref.shape[3]
    base = pl.multiple_of(((length - 1) // 128 - pl.program_id(0)) * 128, 128)
    for h in range(heads):
        w = diag_ref[pl.ds(h, 1), pl.ds(base, _WIN)]
        wb = jnp.broadcast_to(w, (8, _WIN))
        for g in range(tq // 8):
            rolled = pltpu.roll(
                wb, _WIN - 127 + 8 * g, axis=1, stride=1, stride_axis=0
            )
            out_ref[0, h, pl.ds(8 * g, 8), :] = rolled[:, :length]


def kernel(inputs_q, embed_table):
    batch, length, _ = inputs_q.shape
    heads = embed_table.shape[1]

    table_t = jnp.zeros((16, _NUM_BUCKETS), jnp.float32)
    table_t = table_t.at[:heads, :].set(embed_table.T)

    diag_w = ((length - 1) // 128) * 128 + _WIN

    diag = pl.pallas_call(
        functools.partial(_diag_kernel, length=length),
        out_shape=jax.ShapeDtypeStruct((16, diag_w), jnp.float32),
    )(table_t)

    tq = 128
    out = pl.pallas_call(
        functools.partial(_expand_kernel, tq=tq),
        grid=(length // tq,),
        in_specs=[pl.BlockSpec((16, diag_w), lambda i: (0, 0))],
        out_specs=pl.BlockSpec((1, heads, tq, length), lambda i: (0, 0, i, 0)),
        out_shape=jax.ShapeDtypeStruct((batch, heads, length, length), jnp.float32),
        compiler_params=pltpu.CompilerParams(
            dimension_semantics=("arbitrary",)
        ),
    )(diag)
    return out
